# Initial kernel scaffold; baseline (speedup 1.0000x reference)
import jax
import jax.numpy as jnp
from jax.experimental import pallas as pl


def kernel(node_tokens, question_tokens, graph_features, state_vec, node_batch):
    raise NotImplementedError("write your pallas kernel here")


# speedup vs baseline: 5.2839x; 5.2839x over previous
import functools

import jax
import jax.numpy as jnp
from jax import lax
from jax.experimental import pallas as pl


_BN = 2048


def _body(idx_ref, gf_ref, qt_ref, nt_ref, sv_ref, out_ref):
    rs = jnp.sum(nt_ref[...] + sv_ref[...], axis=1)
    table = jnp.sum(gf_ref[...] + qt_ref[...], axis=1)
    idx = idx_ref[0, 0, :]
    onehot = (idx[:, None] == lax.broadcasted_iota(jnp.int32, (idx.shape[0], table.shape[0]), 1))
    gathered = jnp.sum(jnp.where(onehot, table[None, :], 0.0), axis=1)
    out_ref[...] = rs + gathered


def kernel(node_tokens, question_tokens, graph_features, state_vec, node_batch):
    n, d = node_tokens.shape
    b = question_tokens.shape[0]
    nb = (n + _BN - 1) // _BN
    np_ = nb * _BN
    idx = node_batch.astype(jnp.int32)
    idx = jnp.pad(idx, (0, np_ - n)).reshape(nb, 1, _BN)

    out = pl.pallas_call(
        _body,
        grid=(nb,),
        in_specs=[
            pl.BlockSpec((1, 1, _BN), lambda i: (i, 0, 0)),
            pl.BlockSpec((b, d), lambda i: (0, 0)),
            pl.BlockSpec((b, d), lambda i: (0, 0)),
            pl.BlockSpec((_BN, d), lambda i: (i, 0)),
            pl.BlockSpec((_BN, d), lambda i: (i, 0)),
        ],
        out_specs=pl.BlockSpec((_BN,), lambda i: (i,)),
        out_shape=jax.ShapeDtypeStruct((n,), jnp.float32),
    )(idx, graph_features, question_tokens, node_tokens, state_vec)
    return out
